# baseline (speedup 1.0000x reference)
import functools

import jax
import jax.numpy as jnp
from jax import lax
from jax.experimental import pallas as pl
from jax.experimental.pallas import tpu as pltpu
from jax.experimental.pallas import tpu_sc as plsc

_NUM_EMBED = 8192
_EMBED_DIM = 32
_BETA = 0.25
_N_TOK = 8192

_BN = 1024
_BK = 1024


def _argmax_body(z_ref, w_ref, idx_ref, bv_ref, bi_ref):
    k = pl.program_id(1)

    @pl.when(k == 0)
    def _init():
        bv_ref[...] = jnp.full((_BN,), -jnp.inf, jnp.float32)
        bi_ref[...] = jnp.zeros((_BN,), jnp.int32)

    z = z_ref[...]
    zn = z / jnp.maximum(jnp.sqrt(jnp.sum(z * z, axis=1, keepdims=True)), 1e-12)
    w = w_ref[...]
    wn = w / jnp.maximum(jnp.sqrt(jnp.sum(w * w, axis=1, keepdims=True)), 1e-12)
    s = lax.dot_general(zn, wn, (((1,), (1,)), ((), ())),
                        preferred_element_type=jnp.float32)
    m = jnp.max(s, axis=1)
    col = lax.broadcasted_iota(jnp.int32, (_BN, _BK), 1)
    li = jnp.min(jnp.where(s == m[:, None], col, _NUM_EMBED), axis=1)
    gi = k * _BK + li
    upd = m > bv_ref[...]
    bi_ref[...] = jnp.where(upd, gi, bi_ref[...])
    bv_ref[...] = jnp.where(upd, m, bv_ref[...])

    @pl.when(k == pl.num_programs(1) - 1)
    def _flush():
        idx_ref[...] = bi_ref[...]


_argmax_call = pl.pallas_call(
    _argmax_body,
    grid=(_N_TOK // _BN, _NUM_EMBED // _BK),
    in_specs=[
        pl.BlockSpec((_BN, _EMBED_DIM), lambda n, k: (n, 0)),
        pl.BlockSpec((_BK, _EMBED_DIM), lambda n, k: (k, 0)),
    ],
    out_specs=pl.BlockSpec((_BN,), lambda n, k: (n,)),
    out_shape=jax.ShapeDtypeStruct((_N_TOK,), jnp.int32),
    scratch_shapes=[
        pltpu.VMEM((_BN,), jnp.float32),
        pltpu.VMEM((_BN,), jnp.int32),
    ],
)


_NC, _NS = 2, 16
_NW = _NC * _NS
_CH = 128
_CPW = _N_TOK // _NW // _CH
_ZPS = _NUM_EMBED // _NS

@functools.cache
def _sc_gather_hist_call():
    mesh = plsc.VectorSubcoreMesh(core_axis_name="c", subcore_axis_name="s",
                                  num_cores=_NC, num_subcores=_NS)
    return pl.kernel(
        _sc_gather_hist,
        mesh=mesh,
        compiler_params=pltpu.CompilerParams(use_tc_tiling_on_sc=False),
        out_type=[
            jax.ShapeDtypeStruct((_N_TOK // _CH, _CH, _EMBED_DIM), jnp.float32),
            jax.ShapeDtypeStruct((_NC, _NUM_EMBED), jnp.float32),
        ],
        scratch_types=[
            pltpu.VMEM((_CPW, _CH), jnp.int32),
            pltpu.VMEM((_CPW, _CH, _EMBED_DIM), jnp.float32),
            pltpu.VMEM((_CPW * _CH,), jnp.float32),
            pltpu.VMEM((_ZPS,), jnp.float32),
            pltpu.VMEM_SHARED((_NUM_EMBED,), jnp.float32),
            pltpu.SemaphoreType.DMA,
        ],
    )


def _sc_gather_hist(w_hbm, idx_hbm, zq_hbm, cnt_hbm,
                    idx_v, rows_v, ones_v, zeros_v, cnt_sh, sem):
    cid = lax.axis_index("c")
    sid = lax.axis_index("s")
    wid = sid * _NC + cid
    pltpu.sync_copy(idx_hbm.at[pl.ds(wid * _CPW, _CPW)], idx_v)
    cps = []
    for j in range(_CPW):
        cps.append(pltpu.async_copy(w_hbm.at[idx_v.at[j]], rows_v.at[j], sem))

    def _fill_z(i, _):
        zeros_v[pl.ds(i * 16, 16)] = jnp.zeros((16,), jnp.float32)
        return 0

    lax.fori_loop(0, _ZPS // 16, _fill_z, 0)

    def _fill_o(i, _):
        ones_v[pl.ds(i * 16, 16)] = jnp.ones((16,), jnp.float32)
        return 0

    lax.fori_loop(0, (_CPW * _CH) // 16, _fill_o, 0)

    pltpu.sync_copy(zeros_v, cnt_sh.at[pl.ds(sid * _ZPS, _ZPS)])
    for cp in cps:
        cp.wait()
    pltpu.sync_copy(rows_v, zq_hbm.at[pl.ds(wid * _CPW, _CPW)])
    plsc.subcore_barrier()
    for j in range(_CPW):
        pltpu.sync_copy(ones_v.at[pl.ds(j * _CH, _CH)],
                        cnt_sh.at[idx_v.at[j]], add=True)
    plsc.subcore_barrier()

    @pl.when(sid == 0)
    def _flush():
        pltpu.sync_copy(cnt_sh, cnt_hbm.at[cid])


def _finish_body(z_ref, zq_ref, cnt_ref, zqst_ref, loss_ref, perp_ref):
    z = z_ref[...]
    zq = zq_ref[...]
    zqst_ref[...] = z + (zq - z)
    d = zq - z
    m = jnp.sum(d * d) / float(_N_TOK * _EMBED_DIM)
    loss_ref[...] = jnp.full((1, 1), _BETA * m + m, jnp.float32)
    c = cnt_ref[0, :] + cnt_ref[1, :]
    avg = c * (1.0 / _N_TOK)
    ent = jnp.sum(avg * jnp.log(avg + 1e-10))
    perp_ref[...] = jnp.full((1, 1), jnp.exp(-ent), jnp.float32)


_finish_call = pl.pallas_call(
    _finish_body,
    out_shape=[
        jax.ShapeDtypeStruct((_N_TOK, _EMBED_DIM), jnp.float32),
        jax.ShapeDtypeStruct((1, 1), jnp.float32),
        jax.ShapeDtypeStruct((1, 1), jnp.float32),
    ],
)


def kernel(z, W):
    z_flat = z.reshape(_N_TOK, _EMBED_DIM)
    idx = _argmax_call(z_flat, W)
    zq_chunks, cnt = _sc_gather_hist_call()(W, idx.reshape(_N_TOK // _CH, _CH))
    zq = zq_chunks.reshape(_N_TOK, _EMBED_DIM)
    zqst, loss, perp = _finish_call(z_flat, zq, cnt)
    return (zqst.reshape(z.shape), loss.reshape(()), perp.reshape(()), idx)


# speedup vs baseline: 1.5174x; 1.5174x over previous
import functools

import jax
import jax.numpy as jnp
from jax import lax
from jax.experimental import pallas as pl
from jax.experimental.pallas import tpu as pltpu
from jax.experimental.pallas import tpu_sc as plsc

_NUM_EMBED = 8192
_EMBED_DIM = 32
_BETA = 0.25
_N_TOK = 8192

_BN = 256


def _argmax_body(z_ref, w_ref, idx_ref):
    z = z_ref[...]
    zn = z / jnp.maximum(jnp.sqrt(jnp.sum(z * z, axis=1, keepdims=True)), 1e-12)
    w = w_ref[...]
    wn = w / jnp.maximum(jnp.sqrt(jnp.sum(w * w, axis=1, keepdims=True)), 1e-12)
    s = lax.dot_general(zn, wn, (((1,), (1,)), ((), ())),
                        preferred_element_type=jnp.float32)
    idx_ref[...] = jnp.argmax(s, axis=1).astype(jnp.int32)


_argmax_call = pl.pallas_call(
    _argmax_body,
    grid=(_N_TOK // _BN,),
    in_specs=[
        pl.BlockSpec((_BN, _EMBED_DIM), lambda n: (n, 0)),
        pl.BlockSpec((_NUM_EMBED, _EMBED_DIM), lambda n: (0, 0)),
    ],
    out_specs=pl.BlockSpec((_BN,), lambda n: (n,)),
    out_shape=jax.ShapeDtypeStruct((_N_TOK,), jnp.int32),
)


_NC, _NS = 2, 16
_NW = _NC * _NS
_CH = 128
_CPW = _N_TOK // _NW // _CH
_ZPS = _NUM_EMBED // _NS

@functools.cache
def _sc_gather_hist_call():
    mesh = plsc.VectorSubcoreMesh(core_axis_name="c", subcore_axis_name="s",
                                  num_cores=_NC, num_subcores=_NS)
    return pl.kernel(
        _sc_gather_hist,
        mesh=mesh,
        compiler_params=pltpu.CompilerParams(use_tc_tiling_on_sc=False),
        out_type=[
            jax.ShapeDtypeStruct((_N_TOK // _CH, _CH, _EMBED_DIM), jnp.float32),
            jax.ShapeDtypeStruct((_NC, _NUM_EMBED), jnp.float32),
        ],
        scratch_types=[
            pltpu.VMEM((_CPW, _CH), jnp.int32),
            pltpu.VMEM((_CPW, _CH, _EMBED_DIM), jnp.float32),
            pltpu.VMEM((_CPW * _CH,), jnp.float32),
            pltpu.VMEM((_ZPS,), jnp.float32),
            pltpu.VMEM_SHARED((_NUM_EMBED,), jnp.float32),
            pltpu.SemaphoreType.DMA,
        ],
    )


def _sc_gather_hist(w_hbm, idx_hbm, zq_hbm, cnt_hbm,
                    idx_v, rows_v, ones_v, zeros_v, cnt_sh, sem):
    cid = lax.axis_index("c")
    sid = lax.axis_index("s")
    wid = sid * _NC + cid
    pltpu.sync_copy(idx_hbm.at[pl.ds(wid * _CPW, _CPW)], idx_v)
    cps = []
    for j in range(_CPW):
        cps.append(pltpu.async_copy(w_hbm.at[idx_v.at[j]], rows_v.at[j], sem))

    def _fill_z(i, _):
        zeros_v[pl.ds(i * 16, 16)] = jnp.zeros((16,), jnp.float32)
        return 0

    lax.fori_loop(0, _ZPS // 16, _fill_z, 0)

    def _fill_o(i, _):
        ones_v[pl.ds(i * 16, 16)] = jnp.ones((16,), jnp.float32)
        return 0

    lax.fori_loop(0, (_CPW * _CH) // 16, _fill_o, 0)

    pltpu.sync_copy(zeros_v, cnt_sh.at[pl.ds(sid * _ZPS, _ZPS)])
    for cp in cps:
        cp.wait()
    pltpu.sync_copy(rows_v, zq_hbm.at[pl.ds(wid * _CPW, _CPW)])
    plsc.subcore_barrier()
    for j in range(_CPW):
        pltpu.sync_copy(ones_v.at[pl.ds(j * _CH, _CH)],
                        cnt_sh.at[idx_v.at[j]], add=True)
    plsc.subcore_barrier()

    @pl.when(sid == 0)
    def _flush():
        pltpu.sync_copy(cnt_sh, cnt_hbm.at[cid])


def _finish_body(z_ref, zq_ref, cnt_ref, zqst_ref, loss_ref, perp_ref):
    z = z_ref[...]
    zq = zq_ref[...]
    zqst_ref[...] = z + (zq - z)
    d = zq - z
    m = jnp.sum(d * d) / float(_N_TOK * _EMBED_DIM)
    loss_ref[...] = jnp.full((1, 1), _BETA * m + m, jnp.float32)
    c = cnt_ref[0, :] + cnt_ref[1, :]
    avg = c * (1.0 / _N_TOK)
    ent = jnp.sum(avg * jnp.log(avg + 1e-10))
    perp_ref[...] = jnp.full((1, 1), jnp.exp(-ent), jnp.float32)


_finish_call = pl.pallas_call(
    _finish_body,
    out_shape=[
        jax.ShapeDtypeStruct((_N_TOK, _EMBED_DIM), jnp.float32),
        jax.ShapeDtypeStruct((1, 1), jnp.float32),
        jax.ShapeDtypeStruct((1, 1), jnp.float32),
    ],
)


def kernel(z, W):
    z_flat = z.reshape(_N_TOK, _EMBED_DIM)
    idx = _argmax_call(z_flat, W)
    zq_chunks, cnt = _sc_gather_hist_call()(W, idx.reshape(_N_TOK // _CH, _CH))
    zq = zq_chunks.reshape(_N_TOK, _EMBED_DIM)
    zqst, loss, perp = _finish_call(z_flat, zq, cnt)
    return (zqst.reshape(z.shape), loss.reshape(()), perp.reshape(()), idx)


# speedup vs baseline: 1.9643x; 1.2944x over previous
import functools

import jax
import jax.numpy as jnp
from jax import lax
from jax.experimental import pallas as pl
from jax.experimental.pallas import tpu as pltpu
from jax.experimental.pallas import tpu_sc as plsc

_NUM_EMBED = 8192
_EMBED_DIM = 32
_BETA = 0.25
_N_TOK = 8192
_B = 8
_T = 1024

_BN = 256
_NB = _N_TOK // _BN


def _argmax_body(z_ref, w_ref, idx_ref, wn_ref):
    @pl.when(pl.program_id(0) == 0)
    def _norm_w():
        w = w_ref[...]
        wn_ref[...] = w / jnp.maximum(
            jnp.sqrt(jnp.sum(w * w, axis=1, keepdims=True)), 1e-12)

    z = z_ref[...].reshape(_BN, _EMBED_DIM)
    zn = z / jnp.maximum(jnp.sqrt(jnp.sum(z * z, axis=1, keepdims=True)), 1e-12)
    s = lax.dot_general(wn_ref[...], zn, (((1,), (1,)), ((), ())),
                        preferred_element_type=jnp.float32)
    idx_ref[...] = jnp.argmax(s, axis=0).astype(jnp.int32)


_argmax_call = pl.pallas_call(
    _argmax_body,
    grid=(_NB,),
    in_specs=[
        pl.BlockSpec((1, _BN, _EMBED_DIM),
                     lambda n: (n // (_T // _BN), n % (_T // _BN), 0)),
        pl.BlockSpec((_NUM_EMBED, _EMBED_DIM), lambda n: (0, 0)),
    ],
    out_specs=pl.BlockSpec((_BN,), lambda n: (n,)),
    out_shape=jax.ShapeDtypeStruct((_N_TOK,), jnp.int32),
    scratch_shapes=[pltpu.VMEM((_NUM_EMBED, _EMBED_DIM), jnp.float32)],
)


_NC, _NS = 2, 16
_NW = _NC * _NS
_CH = 128
_CPW = _N_TOK // _NW // _CH
_ZPS = _NUM_EMBED // _NS


@functools.cache
def _sc_gather_hist_call():
    mesh = plsc.VectorSubcoreMesh(core_axis_name="c", subcore_axis_name="s",
                                  num_cores=_NC, num_subcores=_NS)
    return pl.kernel(
        _sc_gather_hist,
        mesh=mesh,
        compiler_params=pltpu.CompilerParams(use_tc_tiling_on_sc=False),
        out_type=[
            jax.ShapeDtypeStruct((_N_TOK, _EMBED_DIM), jnp.float32),
            jax.ShapeDtypeStruct((_NC, _NUM_EMBED), jnp.float32),
        ],
        scratch_types=[
            pltpu.VMEM((_CPW, _CH), jnp.int32),
            pltpu.VMEM((_CPW, _CH, _EMBED_DIM), jnp.float32),
            pltpu.VMEM((_CPW * _CH,), jnp.float32),
            pltpu.VMEM((_ZPS,), jnp.float32),
            pltpu.VMEM_SHARED((_NUM_EMBED,), jnp.float32),
            pltpu.SemaphoreType.DMA,
        ],
    )


def _sc_gather_hist(w_hbm, idx_hbm, zq_hbm, cnt_hbm,
                    idx_v, rows_v, ones_v, zeros_v, cnt_sh, sem):
    cid = lax.axis_index("c")
    sid = lax.axis_index("s")
    wid = sid * _NC + cid
    base = wid * _CPW * _CH
    for j in range(_CPW):
        pltpu.sync_copy(idx_hbm.at[pl.ds(base + j * _CH, _CH)], idx_v.at[j])
    cps = []
    for j in range(_CPW):
        cps.append(pltpu.async_copy(w_hbm.at[idx_v.at[j]], rows_v.at[j], sem))

    def _fill_z(i, _):
        zeros_v[pl.ds(i * 16, 16)] = jnp.zeros((16,), jnp.float32)
        return 0

    lax.fori_loop(0, _ZPS // 16, _fill_z, 0)

    def _fill_o(i, _):
        ones_v[pl.ds(i * 16, 16)] = jnp.ones((16,), jnp.float32)
        return 0

    lax.fori_loop(0, (_CPW * _CH) // 16, _fill_o, 0)

    pltpu.sync_copy(zeros_v, cnt_sh.at[pl.ds(sid * _ZPS, _ZPS)])
    for cp in cps:
        cp.wait()
    for j in range(_CPW):
        pltpu.sync_copy(rows_v.at[j], zq_hbm.at[pl.ds(base + j * _CH, _CH)])
    plsc.subcore_barrier()
    for j in range(_CPW):
        pltpu.sync_copy(ones_v.at[pl.ds(j * _CH, _CH)],
                        cnt_sh.at[idx_v.at[j]], add=True)
    plsc.subcore_barrier()

    @pl.when(sid == 0)
    def _flush():
        pltpu.sync_copy(cnt_sh, cnt_hbm.at[cid])


def _finish_body(z_ref, zq_ref, cnt_ref, zqst_ref, loss_ref, perp_ref):
    z = z_ref[...]
    zq = zq_ref[...].reshape(_B, _T, _EMBED_DIM)
    zqst_ref[...] = z + (zq - z)
    d = zq - z
    m = jnp.sum(d * d) / float(_N_TOK * _EMBED_DIM)
    loss_ref[...] = jnp.full((1, 1), _BETA * m + m, jnp.float32)
    c = cnt_ref[0, :] + cnt_ref[1, :]
    avg = c * (1.0 / _N_TOK)
    ent = jnp.sum(avg * jnp.log(avg + 1e-10))
    perp_ref[...] = jnp.full((1, 1), jnp.exp(-ent), jnp.float32)


_finish_call = pl.pallas_call(
    _finish_body,
    out_shape=[
        jax.ShapeDtypeStruct((_B, _T, _EMBED_DIM), jnp.float32),
        jax.ShapeDtypeStruct((1, 1), jnp.float32),
        jax.ShapeDtypeStruct((1, 1), jnp.float32),
    ],
)


def kernel(z, W):
    idx = _argmax_call(z, W)
    zq, cnt = _sc_gather_hist_call()(W, idx)
    zqst, loss, perp = _finish_call(z, zq, cnt)
    return (zqst, loss.reshape(()), perp.reshape(()), idx)


# speedup vs baseline: 1.9982x; 1.0173x over previous
import functools

import jax
import jax.numpy as jnp
from jax import lax
from jax.experimental import pallas as pl
from jax.experimental.pallas import tpu as pltpu
from jax.experimental.pallas import tpu_sc as plsc

_NUM_EMBED = 8192
_EMBED_DIM = 32
_BETA = 0.25
_N_TOK = 8192
_B = 8
_T = 1024

_BN = 256
_NB = _N_TOK // _BN


_KC = 8
_CK = _NUM_EMBED // _KC


def _tree_argmax(v, base):
    idx = lax.broadcasted_iota(jnp.int32, v.shape, 0) + base
    r = v.shape[0]
    while r > 1:
        h = r // 2
        cond = v[h:] > v[:h]
        v = jnp.where(cond, v[h:], v[:h])
        idx = jnp.where(cond, idx[h:], idx[:h])
        r = h
    return v[0], idx[0]


def _argmax_body(z_ref, w_ref, idx_ref, wn_ref):
    @pl.when(pl.program_id(0) == 0)
    def _norm_w():
        w = w_ref[...]
        wn_ref[...] = w / jnp.maximum(
            jnp.sqrt(jnp.sum(w * w, axis=1, keepdims=True)), 1e-12)

    z = z_ref[...].reshape(_BN, _EMBED_DIM)
    zn = z / jnp.maximum(
        jnp.sqrt(jnp.sum(z * z, axis=1, keepdims=True)), 1e-12)
    bv = bi = None
    for kc in range(_KC):
        s_c = lax.dot_general(wn_ref[pl.ds(kc * _CK, _CK), :], zn,
                              (((1,), (1,)), ((), ())),
                              preferred_element_type=jnp.float32)
        v_c, i_c = _tree_argmax(s_c, kc * _CK)
        if kc == 0:
            bv, bi = v_c, i_c
        else:
            upd = v_c > bv
            bv = jnp.where(upd, v_c, bv)
            bi = jnp.where(upd, i_c, bi)
    idx_ref[...] = bi


_argmax_call = pl.pallas_call(
    _argmax_body,
    grid=(_NB,),
    in_specs=[
        pl.BlockSpec((1, _BN, _EMBED_DIM),
                     lambda n: (n // (_T // _BN), n % (_T // _BN), 0)),
        pl.BlockSpec((_NUM_EMBED, _EMBED_DIM), lambda n: (0, 0)),
    ],
    out_specs=pl.BlockSpec((_BN,), lambda n: (n,)),
    out_shape=jax.ShapeDtypeStruct((_N_TOK,), jnp.int32),
    scratch_shapes=[pltpu.VMEM((_NUM_EMBED, _EMBED_DIM), jnp.float32)],
)


_NC, _NS = 2, 16
_NW = _NC * _NS
_CH = 128
_CPW = _N_TOK // _NW // _CH
_ZPS = _NUM_EMBED // _NS


@functools.cache
def _sc_gather_hist_call():
    mesh = plsc.VectorSubcoreMesh(core_axis_name="c", subcore_axis_name="s",
                                  num_cores=_NC, num_subcores=_NS)
    return pl.kernel(
        _sc_gather_hist,
        mesh=mesh,
        compiler_params=pltpu.CompilerParams(use_tc_tiling_on_sc=False),
        out_type=[
            jax.ShapeDtypeStruct((_N_TOK, _EMBED_DIM), jnp.float32),
            jax.ShapeDtypeStruct((_NC, _NUM_EMBED), jnp.float32),
        ],
        scratch_types=[
            pltpu.VMEM((_CPW, _CH), jnp.int32),
            pltpu.VMEM((_CPW, _CH, _EMBED_DIM), jnp.float32),
            pltpu.VMEM((_CPW * _CH,), jnp.float32),
            pltpu.VMEM((_ZPS,), jnp.float32),
            pltpu.VMEM_SHARED((_NUM_EMBED,), jnp.float32),
            pltpu.SemaphoreType.DMA,
        ],
    )


def _sc_gather_hist(w_hbm, idx_hbm, zq_hbm, cnt_hbm,
                    idx_v, rows_v, ones_v, zeros_v, cnt_sh, sem):
    cid = lax.axis_index("c")
    sid = lax.axis_index("s")
    wid = sid * _NC + cid
    base = wid * _CPW * _CH
    for j in range(_CPW):
        pltpu.sync_copy(idx_hbm.at[pl.ds(base + j * _CH, _CH)], idx_v.at[j])
    cps = []
    for j in range(_CPW):
        cps.append(pltpu.async_copy(w_hbm.at[idx_v.at[j]], rows_v.at[j], sem))

    def _fill_z(i, _):
        zeros_v[pl.ds(i * 16, 16)] = jnp.zeros((16,), jnp.float32)
        return 0

    lax.fori_loop(0, _ZPS // 16, _fill_z, 0)

    def _fill_o(i, _):
        ones_v[pl.ds(i * 16, 16)] = jnp.ones((16,), jnp.float32)
        return 0

    lax.fori_loop(0, (_CPW * _CH) // 16, _fill_o, 0)

    pltpu.sync_copy(zeros_v, cnt_sh.at[pl.ds(sid * _ZPS, _ZPS)])
    for cp in cps:
        cp.wait()
    for j in range(_CPW):
        pltpu.sync_copy(rows_v.at[j], zq_hbm.at[pl.ds(base + j * _CH, _CH)])
    plsc.subcore_barrier()
    for j in range(_CPW):
        pltpu.sync_copy(ones_v.at[pl.ds(j * _CH, _CH)],
                        cnt_sh.at[idx_v.at[j]], add=True)
    plsc.subcore_barrier()

    @pl.when(sid == 0)
    def _flush():
        pltpu.sync_copy(cnt_sh, cnt_hbm.at[cid])


def _finish_body(z_ref, zq_ref, cnt_ref, zqst_ref, loss_ref, perp_ref):
    z = z_ref[...]
    zq = zq_ref[...].reshape(_B, _T, _EMBED_DIM)
    zqst_ref[...] = z + (zq - z)
    d = zq - z
    m = jnp.sum(d * d) / float(_N_TOK * _EMBED_DIM)
    loss_ref[...] = jnp.full((1, 1), _BETA * m + m, jnp.float32)
    c = cnt_ref[0, :] + cnt_ref[1, :]
    avg = c * (1.0 / _N_TOK)
    ent = jnp.sum(avg * jnp.log(avg + 1e-10))
    perp_ref[...] = jnp.full((1, 1), jnp.exp(-ent), jnp.float32)


_finish_call = pl.pallas_call(
    _finish_body,
    out_shape=[
        jax.ShapeDtypeStruct((_B, _T, _EMBED_DIM), jnp.float32),
        jax.ShapeDtypeStruct((1, 1), jnp.float32),
        jax.ShapeDtypeStruct((1, 1), jnp.float32),
    ],
)


def kernel(z, W):
    idx = _argmax_call(z, W)
    zq, cnt = _sc_gather_hist_call()(W, idx)
    zqst, loss, perp = _finish_call(z, zq, cnt)
    return (zqst, loss.reshape(()), perp.reshape(()), idx)


# speedup vs baseline: 2.1693x; 1.0856x over previous
import functools

import jax
import jax.numpy as jnp
from jax import lax
from jax.experimental import pallas as pl
from jax.experimental.pallas import tpu as pltpu
from jax.experimental.pallas import tpu_sc as plsc

_NUM_EMBED = 8192
_EMBED_DIM = 32
_BETA = 0.25
_N_TOK = 8192
_B = 8
_T = 1024

_BN = 256
_NB = _N_TOK // _BN


_KC = 32
_CK = _NUM_EMBED // _KC


def _tree_argmax(v, base):
    idx = lax.broadcasted_iota(jnp.int32, v.shape, 0) + base
    r = v.shape[0]
    while r > 1:
        h = r // 2
        cond = v[h:] > v[:h]
        v = jnp.where(cond, v[h:], v[:h])
        idx = jnp.where(cond, idx[h:], idx[:h])
        r = h
    return v[0], idx[0]


def _argmax_body(z_ref, w_ref, idx_ref, wn_ref):
    @pl.when(pl.program_id(0) == 0)
    def _norm_w():
        w = w_ref[...]
        wn_ref[...] = w / jnp.maximum(
            jnp.sqrt(jnp.sum(w * w, axis=1, keepdims=True)), 1e-12)

    z = z_ref[...].reshape(_BN, _EMBED_DIM)
    zn = z / jnp.maximum(
        jnp.sqrt(jnp.sum(z * z, axis=1, keepdims=True)), 1e-12)
    bv = bi = None
    for kc in range(_KC):
        s_c = lax.dot_general(wn_ref[pl.ds(kc * _CK, _CK), :], zn,
                              (((1,), (1,)), ((), ())),
                              preferred_element_type=jnp.float32)
        v_c, i_c = _tree_argmax(s_c, kc * _CK)
        if kc == 0:
            bv, bi = v_c, i_c
        else:
            upd = v_c > bv
            bv = jnp.where(upd, v_c, bv)
            bi = jnp.where(upd, i_c, bi)
    idx_ref[...] = bi


_argmax_call = pl.pallas_call(
    _argmax_body,
    grid=(_NB,),
    in_specs=[
        pl.BlockSpec((1, _BN, _EMBED_DIM),
                     lambda n: (n // (_T // _BN), n % (_T // _BN), 0)),
        pl.BlockSpec((_NUM_EMBED, _EMBED_DIM), lambda n: (0, 0)),
    ],
    out_specs=pl.BlockSpec((_BN,), lambda n: (n,)),
    out_shape=jax.ShapeDtypeStruct((_N_TOK,), jnp.int32),
    scratch_shapes=[pltpu.VMEM((_NUM_EMBED, _EMBED_DIM), jnp.float32)],
)


_NC, _NS = 2, 16
_NW = _NC * _NS
_CH = 128
_CPW = _N_TOK // _NW // _CH
_ZPS = _NUM_EMBED // _NS


@functools.cache
def _sc_gather_hist_call():
    mesh = plsc.VectorSubcoreMesh(core_axis_name="c", subcore_axis_name="s",
                                  num_cores=_NC, num_subcores=_NS)
    return pl.kernel(
        _sc_gather_hist,
        mesh=mesh,
        compiler_params=pltpu.CompilerParams(use_tc_tiling_on_sc=False),
        out_type=[
            jax.ShapeDtypeStruct((_N_TOK, _EMBED_DIM), jnp.float32),
            jax.ShapeDtypeStruct((_NC, _NUM_EMBED), jnp.float32),
        ],
        scratch_types=[
            pltpu.VMEM((_CPW, _CH), jnp.int32),
            pltpu.VMEM((_CPW, _CH, _EMBED_DIM), jnp.float32),
            pltpu.VMEM((_CPW * _CH,), jnp.float32),
            pltpu.VMEM((_ZPS,), jnp.float32),
            pltpu.VMEM_SHARED((_NUM_EMBED,), jnp.float32),
            pltpu.SemaphoreType.DMA,
        ],
    )


def _sc_gather_hist(w_hbm, idx_hbm, zq_hbm, cnt_hbm,
                    idx_v, rows_v, ones_v, zeros_v, cnt_sh, sem):
    cid = lax.axis_index("c")
    sid = lax.axis_index("s")
    wid = sid * _NC + cid
    base = wid * _CPW * _CH
    for j in range(_CPW):
        pltpu.sync_copy(idx_hbm.at[pl.ds(base + j * _CH, _CH)], idx_v.at[j])
    cps = []
    for j in range(_CPW):
        cps.append(pltpu.async_copy(w_hbm.at[idx_v.at[j]], rows_v.at[j], sem))

    def _fill_z(i, _):
        zeros_v[pl.ds(i * 16, 16)] = jnp.zeros((16,), jnp.float32)
        return 0

    lax.fori_loop(0, _ZPS // 16, _fill_z, 0)

    def _fill_o(i, _):
        ones_v[pl.ds(i * 16, 16)] = jnp.ones((16,), jnp.float32)
        return 0

    lax.fori_loop(0, (_CPW * _CH) // 16, _fill_o, 0)

    pltpu.sync_copy(zeros_v, cnt_sh.at[pl.ds(sid * _ZPS, _ZPS)])
    for cp in cps:
        cp.wait()
    for j in range(_CPW):
        pltpu.sync_copy(rows_v.at[j], zq_hbm.at[pl.ds(base + j * _CH, _CH)])
    plsc.subcore_barrier()
    for j in range(_CPW):
        pltpu.sync_copy(ones_v.at[pl.ds(j * _CH, _CH)],
                        cnt_sh.at[idx_v.at[j]], add=True)
    plsc.subcore_barrier()

    @pl.when(sid == 0)
    def _flush():
        pltpu.sync_copy(cnt_sh, cnt_hbm.at[cid])


def _finish_body(z_ref, zq_ref, cnt_ref, zqst_ref, loss_ref, perp_ref):
    z = z_ref[...]
    zq = zq_ref[...].reshape(_B, _T, _EMBED_DIM)
    zqst_ref[...] = z + (zq - z)
    d = zq - z
    m = jnp.sum(d * d) / float(_N_TOK * _EMBED_DIM)
    loss_ref[...] = jnp.full((1, 1), _BETA * m + m, jnp.float32)
    c = cnt_ref[0, :] + cnt_ref[1, :]
    avg = c * (1.0 / _N_TOK)
    ent = jnp.sum(avg * jnp.log(avg + 1e-10))
    perp_ref[...] = jnp.full((1, 1), jnp.exp(-ent), jnp.float32)


_finish_call = pl.pallas_call(
    _finish_body,
    out_shape=[
        jax.ShapeDtypeStruct((_B, _T, _EMBED_DIM), jnp.float32),
        jax.ShapeDtypeStruct((1, 1), jnp.float32),
        jax.ShapeDtypeStruct((1, 1), jnp.float32),
    ],
)


def kernel(z, W):
    idx = _argmax_call(z, W)
    zq, cnt = _sc_gather_hist_call()(W, idx)
    zqst, loss, perp = _finish_call(z, zq, cnt)
    return (zqst, loss.reshape(()), perp.reshape(()), idx)


# speedup vs baseline: 2.3389x; 1.0782x over previous
import functools

import jax
import jax.numpy as jnp
from jax import lax
from jax.experimental import pallas as pl
from jax.experimental.pallas import tpu as pltpu
from jax.experimental.pallas import tpu_sc as plsc

_NUM_EMBED = 8192
_EMBED_DIM = 32
_BETA = 0.25
_N_TOK = 8192
_B = 8
_T = 1024

_BN = 512
_NB = _N_TOK // _BN


_KC = 32
_CK = _NUM_EMBED // _KC


def _tree_argmax(v, base):
    idx = lax.broadcasted_iota(jnp.int32, v.shape, 0) + base
    r = v.shape[0]
    while r > 1:
        h = r // 2
        cond = v[h:] > v[:h]
        v = jnp.where(cond, v[h:], v[:h])
        idx = jnp.where(cond, idx[h:], idx[:h])
        r = h
    return v[0], idx[0]


def _argmax_body(z_ref, w_ref, idx_ref, wn_ref):
    @pl.when(pl.program_id(0) == 0)
    def _norm_w():
        w = w_ref[...]
        wn_ref[...] = w / jnp.maximum(
            jnp.sqrt(jnp.sum(w * w, axis=1, keepdims=True)), 1e-12)

    z = z_ref[...].reshape(_BN, _EMBED_DIM)
    zn = z / jnp.maximum(
        jnp.sqrt(jnp.sum(z * z, axis=1, keepdims=True)), 1e-12)
    bv = bi = None
    for kc in range(_KC):
        s_c = lax.dot_general(wn_ref[pl.ds(kc * _CK, _CK), :], zn,
                              (((1,), (1,)), ((), ())),
                              preferred_element_type=jnp.float32)
        v_c, i_c = _tree_argmax(s_c, kc * _CK)
        if kc == 0:
            bv, bi = v_c, i_c
        else:
            upd = v_c > bv
            bv = jnp.where(upd, v_c, bv)
            bi = jnp.where(upd, i_c, bi)
    idx_ref[...] = bi


_argmax_call = pl.pallas_call(
    _argmax_body,
    grid=(_NB,),
    in_specs=[
        pl.BlockSpec((1, _BN, _EMBED_DIM),
                     lambda n: (n // (_T // _BN), n % (_T // _BN), 0)),
        pl.BlockSpec((_NUM_EMBED, _EMBED_DIM), lambda n: (0, 0)),
    ],
    out_specs=pl.BlockSpec((_BN,), lambda n: (n,)),
    out_shape=jax.ShapeDtypeStruct((_N_TOK,), jnp.int32),
    scratch_shapes=[pltpu.VMEM((_NUM_EMBED, _EMBED_DIM), jnp.float32)],
)


_NC, _NS = 2, 16
_NW = _NC * _NS
_CH = 128
_CPW = _N_TOK // _NW // _CH
_ZPS = _NUM_EMBED // _NS


@functools.cache
def _sc_gather_hist_call():
    mesh = plsc.VectorSubcoreMesh(core_axis_name="c", subcore_axis_name="s",
                                  num_cores=_NC, num_subcores=_NS)
    return pl.kernel(
        _sc_gather_hist,
        mesh=mesh,
        compiler_params=pltpu.CompilerParams(use_tc_tiling_on_sc=False),
        out_type=[
            jax.ShapeDtypeStruct((_N_TOK, _EMBED_DIM), jnp.float32),
            jax.ShapeDtypeStruct((_NC, _NUM_EMBED), jnp.float32),
        ],
        scratch_types=[
            pltpu.VMEM((_CPW, _CH), jnp.int32),
            pltpu.VMEM((_CPW, _CH, _EMBED_DIM), jnp.float32),
            pltpu.VMEM((_CPW * _CH,), jnp.float32),
            pltpu.VMEM((_ZPS,), jnp.float32),
            pltpu.VMEM_SHARED((_NUM_EMBED,), jnp.float32),
            pltpu.SemaphoreType.DMA,
        ],
    )


def _sc_gather_hist(w_hbm, idx_hbm, zq_hbm, cnt_hbm,
                    idx_v, rows_v, ones_v, zeros_v, cnt_sh, sem):
    cid = lax.axis_index("c")
    sid = lax.axis_index("s")
    wid = sid * _NC + cid
    base = wid * _CPW * _CH
    for j in range(_CPW):
        pltpu.sync_copy(idx_hbm.at[pl.ds(base + j * _CH, _CH)], idx_v.at[j])
    cps = []
    for j in range(_CPW):
        cps.append(pltpu.async_copy(w_hbm.at[idx_v.at[j]], rows_v.at[j], sem))

    def _fill_z(i, _):
        zeros_v[pl.ds(i * 16, 16)] = jnp.zeros((16,), jnp.float32)
        return 0

    lax.fori_loop(0, _ZPS // 16, _fill_z, 0)

    def _fill_o(i, _):
        ones_v[pl.ds(i * 16, 16)] = jnp.ones((16,), jnp.float32)
        return 0

    lax.fori_loop(0, (_CPW * _CH) // 16, _fill_o, 0)

    pltpu.sync_copy(zeros_v, cnt_sh.at[pl.ds(sid * _ZPS, _ZPS)])
    for cp in cps:
        cp.wait()
    for j in range(_CPW):
        pltpu.sync_copy(rows_v.at[j], zq_hbm.at[pl.ds(base + j * _CH, _CH)])
    plsc.subcore_barrier()
    for j in range(_CPW):
        pltpu.sync_copy(ones_v.at[pl.ds(j * _CH, _CH)],
                        cnt_sh.at[idx_v.at[j]], add=True)
    plsc.subcore_barrier()

    @pl.when(sid == 0)
    def _flush():
        pltpu.sync_copy(cnt_sh, cnt_hbm.at[cid])


def _finish_body(z_ref, zq_ref, cnt_ref, zqst_ref, loss_ref, perp_ref):
    z = z_ref[...]
    zq = zq_ref[...].reshape(_B, _T, _EMBED_DIM)
    zqst_ref[...] = z + (zq - z)
    d = zq - z
    m = jnp.sum(d * d) / float(_N_TOK * _EMBED_DIM)
    loss_ref[...] = jnp.full((1, 1), _BETA * m + m, jnp.float32)
    c = cnt_ref[0, :] + cnt_ref[1, :]
    avg = c * (1.0 / _N_TOK)
    ent = jnp.sum(avg * jnp.log(avg + 1e-10))
    perp_ref[...] = jnp.full((1, 1), jnp.exp(-ent), jnp.float32)


_finish_call = pl.pallas_call(
    _finish_body,
    out_shape=[
        jax.ShapeDtypeStruct((_B, _T, _EMBED_DIM), jnp.float32),
        jax.ShapeDtypeStruct((1, 1), jnp.float32),
        jax.ShapeDtypeStruct((1, 1), jnp.float32),
    ],
)


def kernel(z, W):
    idx = _argmax_call(z, W)
    zq, cnt = _sc_gather_hist_call()(W, idx)
    zqst, loss, perp = _finish_call(z, zq, cnt)
    return (zqst, loss.reshape(()), perp.reshape(()), idx)


# speedup vs baseline: 2.4074x; 1.0293x over previous
import functools

import jax
import jax.numpy as jnp
from jax import lax
from jax.experimental import pallas as pl
from jax.experimental.pallas import tpu as pltpu
from jax.experimental.pallas import tpu_sc as plsc

_NUM_EMBED = 8192
_EMBED_DIM = 32
_BETA = 0.25
_N_TOK = 8192
_B = 8
_T = 1024

_BN = 1024
_NB = _N_TOK // _BN


_KC = 32
_CK = _NUM_EMBED // _KC


def _tree_argmax(v, base):
    idx = lax.broadcasted_iota(jnp.int32, v.shape, 0) + base
    r = v.shape[0]
    while r > 1:
        h = r // 2
        cond = v[h:] > v[:h]
        v = jnp.where(cond, v[h:], v[:h])
        idx = jnp.where(cond, idx[h:], idx[:h])
        r = h
    return v[0], idx[0]


def _argmax_body(z_ref, w_ref, idx_ref, wn_ref):
    @pl.when(pl.program_id(0) == 0)
    def _norm_w():
        w = w_ref[...]
        wn_ref[...] = w / jnp.maximum(
            jnp.sqrt(jnp.sum(w * w, axis=1, keepdims=True)), 1e-12)

    z = z_ref[...].reshape(_BN, _EMBED_DIM)
    zn = z / jnp.maximum(
        jnp.sqrt(jnp.sum(z * z, axis=1, keepdims=True)), 1e-12)
    bv = bi = None
    for kc in range(_KC):
        s_c = lax.dot_general(wn_ref[pl.ds(kc * _CK, _CK), :], zn,
                              (((1,), (1,)), ((), ())),
                              preferred_element_type=jnp.float32)
        v_c, i_c = _tree_argmax(s_c, kc * _CK)
        if kc == 0:
            bv, bi = v_c, i_c
        else:
            upd = v_c > bv
            bv = jnp.where(upd, v_c, bv)
            bi = jnp.where(upd, i_c, bi)
    idx_ref[...] = bi


_argmax_call = pl.pallas_call(
    _argmax_body,
    grid=(_NB,),
    in_specs=[
        pl.BlockSpec((1, _BN, _EMBED_DIM),
                     lambda n: (n // (_T // _BN), n % (_T // _BN), 0)),
        pl.BlockSpec((_NUM_EMBED, _EMBED_DIM), lambda n: (0, 0)),
    ],
    out_specs=pl.BlockSpec((_BN,), lambda n: (n,)),
    out_shape=jax.ShapeDtypeStruct((_N_TOK,), jnp.int32),
    scratch_shapes=[pltpu.VMEM((_NUM_EMBED, _EMBED_DIM), jnp.float32)],
)


_NC, _NS = 2, 16
_NW = _NC * _NS
_CH = 128
_CPW = _N_TOK // _NW // _CH
_ZPS = _NUM_EMBED // _NS


@functools.cache
def _sc_gather_hist_call():
    mesh = plsc.VectorSubcoreMesh(core_axis_name="c", subcore_axis_name="s",
                                  num_cores=_NC, num_subcores=_NS)
    return pl.kernel(
        _sc_gather_hist,
        mesh=mesh,
        compiler_params=pltpu.CompilerParams(use_tc_tiling_on_sc=False),
        out_type=[
            jax.ShapeDtypeStruct((_N_TOK, _EMBED_DIM), jnp.float32),
            jax.ShapeDtypeStruct((_NC, _NUM_EMBED), jnp.float32),
        ],
        scratch_types=[
            pltpu.VMEM((_CPW, _CH), jnp.int32),
            pltpu.VMEM((_CPW, _CH, _EMBED_DIM), jnp.float32),
            pltpu.VMEM((_CPW * _CH,), jnp.float32),
            pltpu.VMEM((_ZPS,), jnp.float32),
            pltpu.VMEM_SHARED((_NUM_EMBED,), jnp.float32),
            pltpu.SemaphoreType.DMA,
        ],
    )


def _sc_gather_hist(w_hbm, idx_hbm, zq_hbm, cnt_hbm,
                    idx_v, rows_v, ones_v, zeros_v, cnt_sh, sem):
    cid = lax.axis_index("c")
    sid = lax.axis_index("s")
    wid = sid * _NC + cid
    base = wid * _CPW * _CH
    for j in range(_CPW):
        pltpu.sync_copy(idx_hbm.at[pl.ds(base + j * _CH, _CH)], idx_v.at[j])
    cps = []
    for j in range(_CPW):
        cps.append(pltpu.async_copy(w_hbm.at[idx_v.at[j]], rows_v.at[j], sem))

    def _fill_z(i, _):
        zeros_v[pl.ds(i * 16, 16)] = jnp.zeros((16,), jnp.float32)
        return 0

    lax.fori_loop(0, _ZPS // 16, _fill_z, 0)

    def _fill_o(i, _):
        ones_v[pl.ds(i * 16, 16)] = jnp.ones((16,), jnp.float32)
        return 0

    lax.fori_loop(0, (_CPW * _CH) // 16, _fill_o, 0)

    pltpu.sync_copy(zeros_v, cnt_sh.at[pl.ds(sid * _ZPS, _ZPS)])
    for cp in cps:
        cp.wait()
    for j in range(_CPW):
        pltpu.sync_copy(rows_v.at[j], zq_hbm.at[pl.ds(base + j * _CH, _CH)])
    plsc.subcore_barrier()
    for j in range(_CPW):
        pltpu.sync_copy(ones_v.at[pl.ds(j * _CH, _CH)],
                        cnt_sh.at[idx_v.at[j]], add=True)
    plsc.subcore_barrier()

    @pl.when(sid == 0)
    def _flush():
        pltpu.sync_copy(cnt_sh, cnt_hbm.at[cid])


def _finish_body(z_ref, zq_ref, cnt_ref, zqst_ref, loss_ref, perp_ref):
    z = z_ref[...]
    zq = zq_ref[...].reshape(_B, _T, _EMBED_DIM)
    zqst_ref[...] = z + (zq - z)
    d = zq - z
    m = jnp.sum(d * d) / float(_N_TOK * _EMBED_DIM)
    loss_ref[...] = jnp.full((1, 1), _BETA * m + m, jnp.float32)
    c = cnt_ref[0, :] + cnt_ref[1, :]
    avg = c * (1.0 / _N_TOK)
    ent = jnp.sum(avg * jnp.log(avg + 1e-10))
    perp_ref[...] = jnp.full((1, 1), jnp.exp(-ent), jnp.float32)


_finish_call = pl.pallas_call(
    _finish_body,
    out_shape=[
        jax.ShapeDtypeStruct((_B, _T, _EMBED_DIM), jnp.float32),
        jax.ShapeDtypeStruct((1, 1), jnp.float32),
        jax.ShapeDtypeStruct((1, 1), jnp.float32),
    ],
)


def kernel(z, W):
    idx = _argmax_call(z, W)
    zq, cnt = _sc_gather_hist_call()(W, idx)
    zqst, loss, perp = _finish_call(z, zq, cnt)
    return (zqst, loss.reshape(()), perp.reshape(()), idx)
